# Initial kernel scaffold; baseline (speedup 1.0000x reference)
import jax
import jax.numpy as jnp
from jax.experimental import pallas as pl


def kernel(ids, weight):
    raise NotImplementedError("write your pallas kernel here")


# speedup vs baseline: 1.7991x; 1.7991x over previous
import functools

import jax
import jax.numpy as jnp
from jax import lax
from jax.experimental import pallas as pl
from jax.experimental.pallas import tpu as pltpu
from jax.experimental.pallas import tpu_sc as plsc

_NUM_WORKERS = 32
_CHUNK = 2048


@functools.lru_cache(maxsize=None)
def _make_gather(B, D_words, b_per_w, chunk):
    mesh = plsc.VectorSubcoreMesh(core_axis_name="c", subcore_axis_name="s")
    n_chunks = b_per_w // chunk

    @functools.partial(
        pl.kernel,
        mesh=mesh,
        out_type=jax.ShapeDtypeStruct((B, D_words), jnp.int32),
        scratch_types=[
            pltpu.VMEM((chunk,), jnp.int32),
            pltpu.VMEM((chunk, D_words), jnp.int32),
            pltpu.SemaphoreType.DMA,
        ],
        compiler_params=pltpu.CompilerParams(use_tc_tiling_on_sc=False),
    )
    def gather_kernel(idx_hbm, table_hbm, out_hbm, idx_v, rows_v, sem):
        wid = lax.axis_index("s") * 2 + lax.axis_index("c")
        base = wid * b_per_w

        def body(i, carry):
            off = base + i * chunk
            pltpu.sync_copy(idx_hbm.at[pl.ds(off, chunk)], idx_v)
            pltpu.async_copy(table_hbm.at[idx_v], rows_v, sem).wait()
            pltpu.sync_copy(rows_v, out_hbm.at[pl.ds(off, chunk)])
            return carry

        lax.fori_loop(0, n_chunks, body, 0)

    return gather_kernel


def kernel(ids, weight):
    S, T = ids.shape
    V, D = weight.shape
    B = S * T
    Dw = D // 2
    idx = ids.reshape(B).astype(jnp.int32)
    table = lax.bitcast_convert_type(weight.reshape(V, Dw, 2), jnp.int32)
    b_per_w = B // _NUM_WORKERS
    out = _make_gather(B, Dw, b_per_w, _CHUNK)(idx, table)
    return lax.bitcast_convert_type(out, jnp.bfloat16).reshape(S, T, D)


# speedup vs baseline: 1.8237x; 1.0137x over previous
import functools

import jax
import jax.numpy as jnp
from jax import lax
from jax.experimental import pallas as pl
from jax.experimental.pallas import tpu as pltpu
from jax.experimental.pallas import tpu_sc as plsc

_NUM_WORKERS = 32
_CHUNK = 1600
_NBUF = 4


@functools.lru_cache(maxsize=None)
def _make_gather(B, D_words, b_per_w, chunk, nbuf):
    mesh = plsc.VectorSubcoreMesh(core_axis_name="c", subcore_axis_name="s")
    n_chunks = b_per_w // chunk
    n_groups = n_chunks // nbuf

    @functools.partial(
        pl.kernel,
        mesh=mesh,
        out_type=jax.ShapeDtypeStruct((B, D_words), jnp.int32),
        scratch_types=[
            [pltpu.VMEM((chunk,), jnp.int32)] * nbuf,
            [pltpu.VMEM((chunk, D_words), jnp.int32)] * nbuf,
            [pltpu.SemaphoreType.DMA] * nbuf,
            [pltpu.SemaphoreType.DMA] * nbuf,
        ],
        compiler_params=pltpu.CompilerParams(use_tc_tiling_on_sc=False),
    )
    def gather_kernel(idx_hbm, table_hbm, out_hbm, idx_v, rows_v, g_sems, w_sems):
        wid = lax.axis_index("s") * 2 + lax.axis_index("c")
        base = wid * b_per_w

        for b in range(nbuf):
            off = base + b * chunk
            pltpu.sync_copy(idx_hbm.at[pl.ds(off, chunk)], idx_v[b])
            pltpu.async_copy(table_hbm.at[idx_v[b]], rows_v[b], g_sems[b])

        def body(j, carry):
            for b in range(nbuf):
                i = j * nbuf + b
                off = base + i * chunk
                nxt_off = off + nbuf * chunk
                pltpu.make_async_copy(
                    table_hbm.at[idx_v[b]], rows_v[b], g_sems[b]
                ).wait()
                pltpu.async_copy(
                    rows_v[b], out_hbm.at[pl.ds(off, chunk)], w_sems[b]
                )
                pltpu.sync_copy(idx_hbm.at[pl.ds(nxt_off, chunk)], idx_v[b])
                pltpu.make_async_copy(
                    rows_v[b], out_hbm.at[pl.ds(off, chunk)], w_sems[b]
                ).wait()
                pltpu.async_copy(table_hbm.at[idx_v[b]], rows_v[b], g_sems[b])
            return carry

        lax.fori_loop(0, n_groups - 1, body, 0)

        for b in range(nbuf):
            i = (n_groups - 1) * nbuf + b
            off = base + i * chunk
            pltpu.make_async_copy(
                table_hbm.at[idx_v[b]], rows_v[b], g_sems[b]
            ).wait()
            pltpu.async_copy(
                rows_v[b], out_hbm.at[pl.ds(off, chunk)], w_sems[b]
            )
        for b in range(nbuf):
            i = (n_groups - 1) * nbuf + b
            off = base + i * chunk
            pltpu.make_async_copy(
                rows_v[b], out_hbm.at[pl.ds(off, chunk)], w_sems[b]
            ).wait()

    return gather_kernel


def kernel(ids, weight):
    S, T = ids.shape
    V, D = weight.shape
    B = S * T
    Dw = D // 2
    idx = ids.reshape(B).astype(jnp.int32)
    table = lax.bitcast_convert_type(weight.reshape(V, Dw, 2), jnp.int32)
    b_per_w = B // _NUM_WORKERS
    out = _make_gather(B, Dw, b_per_w, _CHUNK, _NBUF)(idx, table)
    return lax.bitcast_convert_type(out, jnp.bfloat16).reshape(S, T, D)


# speedup vs baseline: 4.1587x; 2.2804x over previous
import functools

import jax
import jax.numpy as jnp
from jax import lax
from jax.experimental import pallas as pl
from jax.experimental.pallas import tpu as pltpu
from jax.experimental.pallas import tpu_sc as plsc

_NUM_WORKERS = 32
_RCHUNK = 8
_NBUF = 4


@functools.lru_cache(maxsize=None)
def _make_gather(S, T, V, D, rchunk, nbuf):
    rows_per_w = S // _NUM_WORKERS
    mesh = plsc.VectorSubcoreMesh(core_axis_name="c", subcore_axis_name="s")
    n_chunks = rows_per_w // rchunk
    n_groups = n_chunks // nbuf

    @functools.partial(
        pl.kernel,
        mesh=mesh,
        out_type=jax.ShapeDtypeStruct((S, T, D), jnp.bfloat16),
        scratch_types=[
            [pltpu.VMEM((rchunk * T,), jnp.int32)] * nbuf,
            [pltpu.VMEM((rchunk * T, D), jnp.bfloat16)] * nbuf,
            [pltpu.SemaphoreType.DMA] * nbuf,
            [pltpu.SemaphoreType.DMA] * nbuf,
        ],
        compiler_params=pltpu.CompilerParams(use_tc_tiling_on_sc=False),
    )
    def gather_kernel(ids_hbm, table_hbm, out_hbm, idx_v, rows_v, g_sems, w_sems):
        wid = lax.axis_index("s") * 2 + lax.axis_index("c")
        base = wid * rows_per_w
        def load_idx(b, row0):
            for t in range(rchunk):
                pltpu.sync_copy(ids_hbm.at[row0 + t], idx_v[b].at[pl.ds(t * T, T)])

        def start_wb(b, row0):
            for t in range(rchunk):
                pltpu.async_copy(
                    rows_v[b].at[pl.ds(t * T, T), :], out_hbm.at[row0 + t], w_sems[b]
                )

        def wait_wb(b, row0):
            for t in range(rchunk):
                pltpu.make_async_copy(
                    rows_v[b].at[pl.ds(t * T, T), :], out_hbm.at[row0 + t], w_sems[b]
                ).wait()

        for b in range(nbuf):
            off = base + b * rchunk
            load_idx(b, off)
            pltpu.async_copy(table_hbm.at[idx_v[b]], rows_v[b], g_sems[b])

        def body(j, carry):
            for b in range(nbuf):
                i = j * nbuf + b
                off = base + i * rchunk
                nxt_off = off + nbuf * rchunk
                pltpu.make_async_copy(
                    table_hbm.at[idx_v[b]], rows_v[b], g_sems[b]
                ).wait()
                start_wb(b, off)
                load_idx(b, nxt_off)
                wait_wb(b, off)
                pltpu.async_copy(table_hbm.at[idx_v[b]], rows_v[b], g_sems[b])
            return carry

        lax.fori_loop(0, n_groups - 1, body, 0)

        for b in range(nbuf):
            off = base + ((n_groups - 1) * nbuf + b) * rchunk
            pltpu.make_async_copy(
                table_hbm.at[idx_v[b]], rows_v[b], g_sems[b]
            ).wait()
            start_wb(b, off)
        for b in range(nbuf):
            off = base + ((n_groups - 1) * nbuf + b) * rchunk
            wait_wb(b, off)

    return gather_kernel


def kernel(ids, weight):
    S, T = ids.shape
    V, D = weight.shape
    ids = ids.astype(jnp.int32)
    return _make_gather(S, T, V, D, _RCHUNK, _NBUF)(ids, weight)


# speedup vs baseline: 5.0737x; 1.2200x over previous
import functools

import jax
import jax.numpy as jnp
from jax import lax
from jax.experimental import pallas as pl
from jax.experimental.pallas import tpu as pltpu
from jax.experimental.pallas import tpu_sc as plsc

_NUM_WORKERS = 32
_TSLAB = 40
_TSUB = 4


@functools.lru_cache(maxsize=None)
def _make_sc_gather(S, T, V, D):
    IPW = S // _NUM_WORKERS
    n_slabs = T // _TSLAB
    n_subs = _TSLAB // _TSUB
    rows_per_sub = _TSUB * IPW
    mesh = plsc.VectorSubcoreMesh(core_axis_name="c", subcore_axis_name="s")

    @functools.partial(
        pl.kernel,
        mesh=mesh,
        out_type=jax.ShapeDtypeStruct((T, 2, S // 128, 8, 128), jnp.int32),
        scratch_types=[
            pltpu.VMEM((IPW, _TSLAB), jnp.int32),
            pltpu.VMEM((rows_per_sub,), jnp.int32),
            pltpu.VMEM((rows_per_sub, D), jnp.bfloat16),
            [pltpu.VMEM((2, 4, 8, 128), jnp.int32)] * 2,
            pltpu.SemaphoreType.DMA,
            [pltpu.SemaphoreType.DMA] * 2,
        ],
        compiler_params=pltpu.CompilerParams(
            use_tc_tiling_on_sc=False, needs_layout_passes=False
        ),
    )
    def sc_kernel(ids_hbm, table_hbm, out_hbm, slab_v, gidx_v, rows_v,
                  stgs, gsem, wsems):
        wid = lax.axis_index("s") * 2 + lax.axis_index("c")
        i0 = wid * IPW
        hi0 = wid * (IPW // 128)
        iota = lax.iota(jnp.int32, 16)
        i_dpt = iota // 8
        i_dp8 = lax.rem(iota, 8)

        def do_slab(slab_i, carry):
            pltpu.sync_copy(
                ids_hbm.at[pl.ds(i0, IPW), pl.ds(slab_i * _TSLAB, _TSLAB)],
                slab_v,
            )

            def do_sub(sub, carry2):
                def build(k, c):
                    t_local = sub * _TSUB + k // (IPW // 16)
                    i_base = lax.rem(k, IPW // 16) * 16
                    vals = plsc.load_gather(
                        slab_v, [i_base + iota, iota * 0 + t_local])
                    gidx_v[pl.ds(k * 16, 16)] = vals
                    return c

                lax.fori_loop(0, rows_per_sub // 16, build, 0)
                pltpu.async_copy(table_hbm.at[gidx_v], rows_v, gsem)
                pltpu.make_async_copy(table_hbm.at[gidx_v], rows_v, gsem).wait()

                for tt in range(_TSUB):
                    b = tt % 2
                    t_g = slab_i * _TSLAB + sub * _TSUB + tt
                    if tt >= 2:
                        pltpu.make_async_copy(
                            stgs[b],
                            out_hbm.at[t_g - 2, :, pl.ds(hi0, 4), :, :],
                            wsems[b],
                        ).wait()

                    def xpose(r, c):
                        v32 = rows_v[tt * IPW + r, :]
                        w16 = plsc.bitcast(v32, jnp.int32)
                        hi = r // 128
                        il = lax.rem(r, 128)
                        plsc.store_scatter(
                            stgs[b],
                            [i_dpt, iota * 0 + hi, i_dp8, iota * 0 + il],
                            w16,
                        )
                        return c

                    lax.fori_loop(0, IPW, xpose, 0)
                    pltpu.async_copy(
                        stgs[b], out_hbm.at[t_g, :, pl.ds(hi0, 4), :, :],
                        wsems[b],
                    )
                for tt in range(2, 4):
                    b = tt % 2
                    t_g = slab_i * _TSLAB + sub * _TSUB + tt
                    pltpu.make_async_copy(
                        stgs[b], out_hbm.at[t_g, :, pl.ds(hi0, 4), :, :],
                        wsems[b],
                    ).wait()
                return carry2

            lax.fori_loop(0, n_subs, do_sub, 0)
            return carry

        lax.fori_loop(0, n_slabs, do_slab, 0)

    return sc_kernel


@functools.lru_cache(maxsize=None)
def _make_tc_pairsplit(S, T, D):
    def body(x_ref, z_ref):
        x = x_ref[0]
        xu = lax.bitcast_convert_type(x, jnp.uint32)
        lo = (xu & jnp.uint32(0xFFFF)).astype(jnp.uint16)
        hi = (xu >> jnp.uint32(16)).astype(jnp.uint16)
        y = jnp.concatenate([lo[:, None, :], hi[:, None, :]], axis=1)
        z_ref[0] = lax.bitcast_convert_type(y.reshape(D, S), jnp.bfloat16)

    return pl.pallas_call(
        body,
        grid=(T,),
        in_specs=[pl.BlockSpec((1, D // 2, S), lambda i: (i, 0, 0))],
        out_specs=pl.BlockSpec((1, D, S), lambda i: (i, 0, 0)),
        out_shape=jax.ShapeDtypeStruct((T, D, S), jnp.bfloat16),
    )


def kernel(ids, weight):
    S, T = ids.shape
    V, D = weight.shape
    ids = ids.astype(jnp.int32)
    o = _make_sc_gather(S, T, V, D)(ids, weight)
    oo = o.transpose(0, 1, 3, 2, 4).reshape(T, D // 2, S)
    z = _make_tc_pairsplit(S, T, D)(oo)
    return jnp.transpose(z, (2, 0, 1))


# speedup vs baseline: 5.2543x; 1.0356x over previous
import functools

import jax
import jax.numpy as jnp
from jax import lax
from jax.experimental import pallas as pl
from jax.experimental.pallas import tpu as pltpu
from jax.experimental.pallas import tpu_sc as plsc

_NUM_WORKERS = 32
_TSLAB = 40
_TSUB = 4


@functools.lru_cache(maxsize=None)
def _make_sc_gather(S, T, V, D):
    IPW = S // _NUM_WORKERS
    n_slabs = T // _TSLAB
    n_subs = _TSLAB // _TSUB
    n_pairs = n_subs // 2
    rows_per_sub = _TSUB * IPW
    vpt = IPW // 16
    mesh = plsc.VectorSubcoreMesh(core_axis_name="c", subcore_axis_name="s")

    @functools.partial(
        pl.kernel,
        mesh=mesh,
        out_type=jax.ShapeDtypeStruct((T, 2, S // 128, 8, 128), jnp.int32),
        scratch_types=[
            pltpu.VMEM((IPW, _TSLAB), jnp.int32),
            [pltpu.VMEM((rows_per_sub,), jnp.int32)] * 2,
            [pltpu.VMEM((rows_per_sub, D), jnp.bfloat16)] * 2,
            [pltpu.VMEM((2, 4, 8, 128), jnp.int32)] * 2,
            [pltpu.SemaphoreType.DMA] * 2,
            [pltpu.SemaphoreType.DMA] * 2,
        ],
        compiler_params=pltpu.CompilerParams(
            use_tc_tiling_on_sc=False, needs_layout_passes=False
        ),
    )
    def sc_kernel(ids_hbm, table_hbm, out_hbm, slab_v, gidxs, rowss,
                  stgs, gsems, wsems):
        wid = lax.axis_index("s") * 2 + lax.axis_index("c")
        i0 = wid * IPW
        hi0 = wid * (IPW // 128)
        iota = lax.iota(jnp.int32, 16)
        i_dpt = iota // 8
        i_dp8 = lax.rem(iota, 8)

        def build(g, sub_local):
            def bloop(k4, c):
                for u in range(4):
                    k = k4 * 4 + u
                    t_local = sub_local * _TSUB + k // vpt
                    i_base = lax.rem(k, vpt) * 16
                    vals = plsc.load_gather(
                        slab_v, [i_base + iota, iota * 0 + t_local])
                    gidxs[g][pl.ds(k * 16, 16)] = vals
                return c

            lax.fori_loop(0, rows_per_sub // 16 // 4, bloop, 0)

        def gather_start(g):
            pltpu.async_copy(table_hbm.at[gidxs[g]], rowss[g], gsems[g])

        def gather_wait(g):
            pltpu.make_async_copy(
                table_hbm.at[gidxs[g]], rowss[g], gsems[g]).wait()

        def drain(g, sub_global):
            for tt in range(_TSUB):
                b = tt % 2
                t_g = sub_global * _TSUB + tt
                if tt >= 2:
                    pltpu.make_async_copy(
                        stgs[b], out_hbm.at[t_g - 2, :, pl.ds(hi0, 4), :, :],
                        wsems[b],
                    ).wait()

                def xpose(r8, c):
                    hi = (r8 * 8) // 128
                    il_base = lax.rem(r8 * 8, 128)
                    for u in range(8):
                        r = r8 * 8 + u
                        v32 = rowss[g][tt * IPW + r, :]
                        w16 = plsc.bitcast(v32, jnp.int32)
                        plsc.store_scatter(
                            stgs[b],
                            [i_dpt, iota * 0 + hi, i_dp8,
                             iota * 0 + (il_base + u)],
                            w16,
                        )
                    return c

                lax.fori_loop(0, IPW // 8, xpose, 0)
                pltpu.async_copy(
                    stgs[b], out_hbm.at[t_g, :, pl.ds(hi0, 4), :, :],
                    wsems[b],
                )
            for tt in range(2, 4):
                b = tt % 2
                t_g = sub_global * _TSUB + tt
                pltpu.make_async_copy(
                    stgs[b], out_hbm.at[t_g, :, pl.ds(hi0, 4), :, :],
                    wsems[b],
                ).wait()

        def do_slab(slab_i, carry):
            pltpu.sync_copy(
                ids_hbm.at[pl.ds(i0, IPW), pl.ds(slab_i * _TSLAB, _TSLAB)],
                slab_v,
            )
            t_base = slab_i * n_subs

            def do_pair(sp, carry2):
                build(0, 2 * sp)
                gather_start(0)
                build(1, 2 * sp + 1)
                gather_start(1)
                gather_wait(0)
                drain(0, t_base + 2 * sp)
                gather_wait(1)
                drain(1, t_base + 2 * sp + 1)
                return carry2

            lax.fori_loop(0, n_pairs, do_pair, 0)
            return carry

        lax.fori_loop(0, n_slabs, do_slab, 0)

    return sc_kernel


@functools.lru_cache(maxsize=None)
def _make_tc_pairsplit(S, T, D):
    def body(x_ref, z_ref):
        x = x_ref[0]
        xu = lax.bitcast_convert_type(x, jnp.uint32)
        lo = (xu & jnp.uint32(0xFFFF)).astype(jnp.uint16)
        hi = (xu >> jnp.uint32(16)).astype(jnp.uint16)
        y = jnp.concatenate([lo[:, None, :], hi[:, None, :]], axis=1)
        z_ref[0] = lax.bitcast_convert_type(y.reshape(D, S), jnp.bfloat16)

    return pl.pallas_call(
        body,
        grid=(T,),
        in_specs=[pl.BlockSpec((1, D // 2, S), lambda i: (i, 0, 0))],
        out_specs=pl.BlockSpec((1, D, S), lambda i: (i, 0, 0)),
        out_shape=jax.ShapeDtypeStruct((T, D, S), jnp.bfloat16),
    )


def kernel(ids, weight):
    S, T = ids.shape
    V, D = weight.shape
    ids = ids.astype(jnp.int32)
    o = _make_sc_gather(S, T, V, D)(ids, weight)
    oo = o.transpose(0, 1, 3, 2, 4).reshape(T, D // 2, S)
    z = _make_tc_pairsplit(S, T, D)(oo)
    return jnp.transpose(z, (2, 0, 1))


# speedup vs baseline: 7.3628x; 1.4013x over previous
import functools

import jax
import jax.numpy as jnp
from jax import lax
from jax.experimental import pallas as pl
from jax.experimental.pallas import tpu as pltpu
from jax.experimental.pallas import tpu_sc as plsc

_NUM_WORKERS = 32
_TSLAB = 40
_TSUB = 4


@functools.lru_cache(maxsize=None)
def _make_sc_gather(S, T, V, D):
    IPW = S // _NUM_WORKERS
    n_slabs = T // _TSLAB
    n_subs = _TSLAB // _TSUB
    n_pairs = n_subs // 2
    rows_per_sub = _TSUB * IPW
    vpt = IPW // 16
    mesh = plsc.VectorSubcoreMesh(core_axis_name="c", subcore_axis_name="s")

    @functools.partial(
        pl.kernel,
        mesh=mesh,
        out_type=jax.ShapeDtypeStruct((T, 2, S // 128, 8, 128), jnp.int32),
        scratch_types=[
            pltpu.VMEM((IPW, _TSLAB), jnp.int32),
            [pltpu.VMEM((rows_per_sub,), jnp.int32)] * 2,
            [pltpu.VMEM((rows_per_sub, D), jnp.bfloat16)] * 2,
            [pltpu.VMEM((2, 5, 8, 129), jnp.int32)] * 2,
            [pltpu.SemaphoreType.DMA] * 2,
            [pltpu.SemaphoreType.DMA] * 2,
        ],
        compiler_params=pltpu.CompilerParams(
            use_tc_tiling_on_sc=False, needs_layout_passes=False
        ),
    )
    def sc_kernel(ids_hbm, table_hbm, out_hbm, slab_v, gidxs, rowss,
                  stgs, gsems, wsems):
        wid = lax.axis_index("s") * 2 + lax.axis_index("c")
        i0 = wid * IPW
        hi0 = wid * (IPW // 128)
        iota = lax.iota(jnp.int32, 16)
        i_dpt = iota // 8
        i_dp8 = lax.rem(iota, 8)

        def build(g, sub_local):
            def bloop(k4, c):
                for u in range(4):
                    k = k4 * 4 + u
                    t_local = sub_local * _TSUB + k // vpt
                    i_base = lax.rem(k, vpt) * 16
                    vals = plsc.load_gather(
                        slab_v, [i_base + iota, iota * 0 + t_local])
                    gidxs[g][pl.ds(k * 16, 16)] = vals
                return c

            lax.fori_loop(0, rows_per_sub // 16 // 4, bloop, 0)

        def gather_start(g):
            pltpu.async_copy(table_hbm.at[gidxs[g]], rowss[g], gsems[g])

        def gather_wait(g):
            pltpu.make_async_copy(
                table_hbm.at[gidxs[g]], rowss[g], gsems[g]).wait()

        def drain(g, sub_global):
            for tt in range(_TSUB):
                b = tt % 2
                t_g = sub_global * _TSUB + tt
                if tt >= 2:
                    pltpu.make_async_copy(
                        stgs[b].at[:, pl.ds(0, 4), :, pl.ds(0, 128)],
                        out_hbm.at[t_g - 2, :, pl.ds(hi0, 4), :, :],
                        wsems[b],
                    ).wait()

                def xpose(r8, c):
                    hi = (r8 * 8) // 128
                    il_base = lax.rem(r8 * 8, 128)
                    for u in range(8):
                        r = r8 * 8 + u
                        v32 = rowss[g][tt * IPW + r, :]
                        w16 = plsc.bitcast(v32, jnp.int32)
                        plsc.store_scatter(
                            stgs[b],
                            [i_dpt, iota * 0 + hi, i_dp8,
                             iota * 0 + (il_base + u)],
                            w16,
                        )
                    return c

                lax.fori_loop(0, IPW // 8, xpose, 0)
                pltpu.async_copy(
                    stgs[b].at[:, pl.ds(0, 4), :, pl.ds(0, 128)],
                    out_hbm.at[t_g, :, pl.ds(hi0, 4), :, :],
                    wsems[b],
                )
            for tt in range(2, 4):
                b = tt % 2
                t_g = sub_global * _TSUB + tt
                pltpu.make_async_copy(
                    stgs[b].at[:, pl.ds(0, 4), :, pl.ds(0, 128)],
                    out_hbm.at[t_g, :, pl.ds(hi0, 4), :, :],
                    wsems[b],
                ).wait()

        def do_slab(slab_i, carry):
            pltpu.sync_copy(
                ids_hbm.at[pl.ds(i0, IPW), pl.ds(slab_i * _TSLAB, _TSLAB)],
                slab_v,
            )
            t_base = slab_i * n_subs

            def do_pair(sp, carry2):
                build(0, 2 * sp)
                gather_start(0)
                build(1, 2 * sp + 1)
                gather_start(1)
                gather_wait(0)
                drain(0, t_base + 2 * sp)
                gather_wait(1)
                drain(1, t_base + 2 * sp + 1)
                return carry2

            lax.fori_loop(0, n_pairs, do_pair, 0)
            return carry

        lax.fori_loop(0, n_slabs, do_slab, 0)

    return sc_kernel


@functools.lru_cache(maxsize=None)
def _make_tc_pairsplit(S, T, D):
    def body(x_ref, z_ref):
        x = x_ref[0]
        xu = lax.bitcast_convert_type(x, jnp.uint32)
        lo = (xu & jnp.uint32(0xFFFF)).astype(jnp.uint16)
        hi = (xu >> jnp.uint32(16)).astype(jnp.uint16)
        y = jnp.concatenate([lo[:, None, :], hi[:, None, :]], axis=1)
        z_ref[0] = lax.bitcast_convert_type(y.reshape(D, S), jnp.bfloat16)

    return pl.pallas_call(
        body,
        grid=(T,),
        in_specs=[pl.BlockSpec((1, D // 2, S), lambda i: (i, 0, 0))],
        out_specs=pl.BlockSpec((1, D, S), lambda i: (i, 0, 0)),
        out_shape=jax.ShapeDtypeStruct((T, D, S), jnp.bfloat16),
    )


def kernel(ids, weight):
    S, T = ids.shape
    V, D = weight.shape
    ids = ids.astype(jnp.int32)
    o = _make_sc_gather(S, T, V, D)(ids, weight)
    oo = o.transpose(0, 1, 3, 2, 4).reshape(T, D // 2, S)
    z = _make_tc_pairsplit(S, T, D)(oo)
    return jnp.transpose(z, (2, 0, 1))
